# baseline (speedup 1.0000x reference)
import functools

import jax
import jax.numpy as jnp
from jax import lax
from jax.experimental import pallas as pl
from jax.experimental.pallas import tpu as pltpu
from jax.experimental.pallas import tpu_sc as plsc

N = 10000
D = 128
E = 320000

NC = 2
NS = 16
NW = NC * NS
C = 64
NCHUNK = 160
EPW = C * NCHUNK
EP = EPW * NW
NP = 10112
RPT = NP // NS
IB = 4
NB = NCHUNK // IB
NPAIR = NB // 2
NRB = 4

_mesh = plsc.VectorSubcoreMesh(core_axis_name="c", subcore_axis_name="s")


def _make_agg(with_counts: bool):
    out_type = [jax.ShapeDtypeStruct((NC, NP, D), jnp.float32),
                jax.ShapeDtypeStruct((NC, NP, D), jnp.float32)]
    scratch = (
        [pltpu.VMEM((IB, C), jnp.int32) for _ in range(2)] +
        [pltpu.VMEM((IB, C), jnp.int32) for _ in range(2)] +
        [pltpu.VMEM((C, D), jnp.float32) for _ in range(NRB)] +
        [pltpu.SemaphoreType.DMA for _ in range(NRB)] +
        [pltpu.SemaphoreType.DMA for _ in range(NRB)] +
        [pltpu.SemaphoreType.DMA for _ in range(2)] +
        [pltpu.VMEM_SHARED((NP, D), jnp.float32)]
    )
    if with_counts:
        out_type += [jax.ShapeDtypeStruct((NC, NP, 16), jnp.float32),
                     jax.ShapeDtypeStruct((NC, NP, 16), jnp.float32)]
        scratch += [
            pltpu.VMEM((C, 16), jnp.float32),
            pltpu.VMEM((C, 16), jnp.float32),
            pltpu.VMEM_SHARED((NP, 16), jnp.float32),
        ]

    @functools.partial(pl.kernel, out_type=out_type, mesh=_mesh,
                       scratch_types=scratch,
                       compiler_params=pltpu.CompilerParams(
                           use_tc_tiling_on_sc=False))
    def agg(xa_hbm, srca_hbm, dsta_hbm, xb_hbm, srcb_hbm, dstb_hbm, *r):
        if with_counts:
            out_sums = r[0:2]
            out_cnts = r[2:4]
            r = r[4:]
        else:
            out_sums = r[0:2]
            r = r[2:]
        src_i = r[0:2]
        dst_i = r[2:4]
        rows = r[4:4 + NRB]
        gsem = r[4 + NRB:4 + 2 * NRB]
        ssem = r[4 + 2 * NRB:4 + 3 * NRB]
        isem = r[4 + 3 * NRB:6 + 3 * NRB]
        acc_s = r[6 + 3 * NRB]
        if with_counts:
            ones_v, zb_v, cnt_s = r[7 + 3 * NRB:]
        cid = lax.axis_index("c")
        sid = lax.axis_index("s")
        wid = cid * NS + sid

        nfull = RPT // C
        rem = RPT - nfull * C

        def zero_stripes():
            for k in range(nfull):
                pltpu.sync_copy(rows[0],
                                acc_s.at[pl.ds(sid * RPT + k * C, C)])
            if rem:
                pltpu.sync_copy(rows[0].at[pl.ds(0, rem)],
                                acc_s.at[pl.ds(sid * RPT + nfull * C, rem)])
            if with_counts:
                for k in range(nfull):
                    pltpu.sync_copy(zb_v,
                                    cnt_s.at[pl.ds(sid * RPT + k * C, C)])
                if rem:
                    pltpu.sync_copy(
                        zb_v.at[pl.ds(0, rem)],
                        cnt_s.at[pl.ds(sid * RPT + nfull * C, rem)])

        def writeback(rel):
            pltpu.sync_copy(acc_s.at[pl.ds(sid * RPT, RPT)],
                            out_sums[rel].at[cid, pl.ds(sid * RPT, RPT)])
            if with_counts:
                pltpu.sync_copy(cnt_s.at[pl.ds(sid * RPT, RPT)],
                                out_cnts[rel].at[cid, pl.ds(sid * RPT, RPT)])

        def make_ops(x_hbm, src_hbm, dst_hbm):
            def fetch_idx(batch, par):
                pltpu.async_copy(src_hbm.at[wid, pl.ds(batch * IB, IB)],
                                 src_i[par], isem[par])
                pltpu.async_copy(dst_hbm.at[wid, pl.ds(batch * IB, IB)],
                                 dst_i[par], isem[par])

            def wait_idx(par):
                pltpu.make_async_copy(src_hbm.at[wid, pl.ds(0, IB)],
                                      src_i[par], isem[par]).wait()
                pltpu.make_async_copy(dst_hbm.at[wid, pl.ds(0, IB)],
                                      dst_i[par], isem[par]).wait()

            def fire_g(jp):
                pltpu.async_copy(
                    x_hbm.at[src_i[(jp // IB) % 2].at[jp % IB]],
                    rows[jp % NRB], gsem[jp % NRB])

            def wait_g(jp):
                pltpu.make_async_copy(
                    x_hbm.at[src_i[(jp // IB) % 2].at[jp % IB]],
                    rows[jp % NRB], gsem[jp % NRB]).wait()

            def fire_s(jp):
                idx = dst_i[(jp // IB) % 2].at[jp % IB]
                pltpu.async_copy(rows[jp % NRB], acc_s.at[idx],
                                 ssem[jp % NRB], add=True)
                if with_counts:
                    pltpu.async_copy(ones_v, cnt_s.at[idx],
                                     ssem[jp % NRB], add=True)

            def wait_s(jp):
                pltpu.make_async_copy(rows[jp % NRB],
                                      acc_s.at[dst_i[0].at[0]],
                                      ssem[jp % NRB]).wait()
                if with_counts:
                    pltpu.make_async_copy(ones_v, cnt_s.at[dst_i[0].at[0]],
                                          ssem[jp % NRB]).wait()

            def prologue(skip0=False):
                fetch_idx(0, 0)
                wait_idx(0)
                if not skip0:
                    fire_g(0)
                fire_g(1)
                fire_g(2)

            def pair(mm, first, last):
                for jj in range(8):
                    wait_g(jj)
                    fire_s(jj)
                    if not (first and jj == 0):
                        wait_s((jj - 1) % 8)
                    if jj == 0:
                        fetch_idx(2 * mm + 1, 1)
                    if jj == 1:
                        wait_idx(1)
                    if jj == 4 and not last:
                        fetch_idx(2 * mm + 2, 0)
                    if jj == 5 and not last:
                        wait_idx(0)
                    if jj + 3 < 8:
                        fire_g(jj + 3)
                    elif not last:
                        fire_g(jj - 5)

            def run():
                pair(0, True, False)

                def pair_loop(mm, _):
                    pair(mm, False, False)
                    return 0
                lax.fori_loop(1, NPAIR - 1, pair_loop, 0)
                pair(NPAIR - 1, False, True)
                wait_s(7)

            return prologue, fire_g, run

        prologue_a, fire_g_a, run_a = make_ops(xa_hbm, srca_hbm, dsta_hbm)
        prologue_b, fire_g_b, run_b = make_ops(xb_hbm, srcb_hbm, dstb_hbm)

        z16 = jnp.zeros((16,), jnp.float32)

        def zrow(i, _):
            for j in range(D // 16):
                rows[0][i, pl.ds(j * 16, 16)] = z16
            return 0
        lax.fori_loop(0, C, zrow, 0)
        if with_counts:
            o16 = jnp.ones((16,), jnp.float32)

            def frow(i, _):
                ones_v[i, :] = o16
                zb_v[i, :] = z16
                return 0
            lax.fori_loop(0, C, frow, 0)

        zero_stripes()
        prologue_a()
        plsc.subcore_barrier()
        run_a()
        lax.fori_loop(0, C, zrow, 0)
        prologue_b(skip0=True)
        plsc.subcore_barrier()
        writeback(0)
        zero_stripes()
        fire_g_b(0)
        plsc.subcore_barrier()
        run_b()
        plsc.subcore_barrier()
        writeback(1)

    return agg


_agg_counts = _make_agg(True)
_agg_plain = _make_agg(False)

BN = 1000


def _make_dense(relu: bool):
    def body(p_ref, c_ref, x_ref, wl_ref, b_ref, wr_ref, o_ref):
        s = p_ref[0] + p_ref[1]
        cnt = jnp.maximum(c_ref[0, :, 0:1] + c_ref[1, :, 0:1], 1.0)
        agg = s / cnt
        y = jnp.dot(agg, wl_ref[...], preferred_element_type=jnp.float32)
        y = y + jnp.dot(x_ref[...], wr_ref[...],
                        preferred_element_type=jnp.float32)
        y = y + b_ref[...]
        if relu:
            y = jnp.maximum(y, 0.0)
        o_ref[...] = y

    return pl.pallas_call(
        body,
        grid=(N // BN,),
        in_specs=[
            pl.BlockSpec((NC, BN, D), lambda i: (0, i, 0)),
            pl.BlockSpec((NC, BN, 16), lambda i: (0, i, 0)),
            pl.BlockSpec((BN, D), lambda i: (i, 0)),
            pl.BlockSpec((D, D), lambda i: (0, 0)),
            pl.BlockSpec((1, D), lambda i: (0, 0)),
            pl.BlockSpec((D, D), lambda i: (0, 0)),
        ],
        out_specs=pl.BlockSpec((BN, D), lambda i: (i, 0)),
        out_shape=jax.ShapeDtypeStruct((N, D), jnp.float32),
    )


_dense_relu = _make_dense(True)
_dense_out = _make_dense(False)


def _pad_edges(edge_index):
    src, dst = edge_index[0], edge_index[1]
    pad = EP - E
    ar = jnp.arange(pad, dtype=jnp.int32)
    pad_src = (ar * 37) % N
    pad_dst = N + ar % (NP - N)
    src_p = jnp.concatenate([src, pad_src]).reshape(NW, NCHUNK, C)
    dst_p = jnp.concatenate([dst, pad_dst]).reshape(NW, NCHUNK, C)
    return src_p, dst_p


def kernel(x_user, x_item, edge_index_u2i, edge_index_i2u,
           W1l_u2i, b1l_u2i, W1r_u2i, W1l_i2u, b1l_i2u, W1r_i2u,
           W2l_u2i, b2l_u2i, W2r_u2i, W2l_i2u, b2l_i2u, W2r_i2u):
    src_u2i, dst_u2i = _pad_edges(edge_index_u2i)
    src_i2u, dst_i2u = _pad_edges(edge_index_i2u)
    b1l_u2i = b1l_u2i.reshape(1, D)
    b1l_i2u = b1l_i2u.reshape(1, D)
    b2l_u2i = b2l_u2i.reshape(1, D)
    b2l_i2u = b2l_i2u.reshape(1, D)

    sums1_i, sums1_u, cnts_i, cnts_u = _agg_counts(
        x_user, src_u2i, dst_u2i, x_item, src_i2u, dst_i2u)
    h_item = _dense_relu(sums1_i, cnts_i, x_item, W1l_u2i, b1l_u2i, W1r_u2i)
    h_user = _dense_relu(sums1_u, cnts_u, x_user, W1l_i2u, b1l_i2u, W1r_i2u)
    sums2_i, sums2_u = _agg_plain(
        h_user, src_u2i, dst_u2i, h_item, src_i2u, dst_i2u)
    o_item = _dense_out(sums2_i, cnts_i, h_item, W2l_u2i, b2l_u2i, W2r_u2i)
    o_user = _dense_out(sums2_u, cnts_u, h_user, W2l_i2u, b2l_i2u, W2r_i2u)
    return (o_user, o_item)


# speedup vs baseline: 1.0604x; 1.0604x over previous
import functools

import jax
import jax.numpy as jnp
from jax import lax
from jax.experimental import pallas as pl
from jax.experimental.pallas import tpu as pltpu
from jax.experimental.pallas import tpu_sc as plsc

N = 10000
D = 128
E = 320000

NC = 2
NS = 16
NW = NC * NS
C = 64
NCHUNK = 160
EPW = C * NCHUNK
EP = EPW * NW
NP = 10112
RPT = NP // NS
IB = 4
NB = NCHUNK // IB
NPAIR = NB // 2
NRB = 4

_mesh = plsc.VectorSubcoreMesh(core_axis_name="c", subcore_axis_name="s")


def _make_agg(with_counts: bool):
    out_type = [jax.ShapeDtypeStruct((NC, NP, D), jnp.float32)]
    scratch = (
        [pltpu.VMEM((IB, C), jnp.int32) for _ in range(2)] +
        [pltpu.VMEM((IB, C), jnp.int32) for _ in range(2)] +
        [pltpu.VMEM((C, D), jnp.float32) for _ in range(NRB)] +
        [pltpu.SemaphoreType.DMA for _ in range(NRB)] +
        [pltpu.SemaphoreType.DMA for _ in range(NRB)] +
        [pltpu.SemaphoreType.DMA for _ in range(2)] +
        [pltpu.VMEM_SHARED((NP, D), jnp.float32)]
    )
    if with_counts:
        out_type.append(jax.ShapeDtypeStruct((NC, NP, 16), jnp.float32))
        scratch += [
            pltpu.VMEM((C, 16), jnp.float32),
            pltpu.VMEM((C, 16), jnp.float32),
            pltpu.VMEM_SHARED((NP, 16), jnp.float32),
        ]

    @functools.partial(pl.kernel, out_type=out_type, mesh=_mesh,
                       scratch_types=scratch,
                       compiler_params=pltpu.CompilerParams(
                           use_tc_tiling_on_sc=False))
    def agg(x_hbm, src_hbm, dst_hbm, *r):
        if with_counts:
            out_sums, out_cnts = r[0], r[1]
            r = r[2:]
        else:
            out_sums = r[0]
            r = r[1:]
        src_i = r[0:2]
        dst_i = r[2:4]
        rows = r[4:4 + NRB]
        gsem = r[4 + NRB:4 + 2 * NRB]
        ssem = r[4 + 2 * NRB:4 + 3 * NRB]
        isem = r[4 + 3 * NRB:6 + 3 * NRB]
        acc_s = r[6 + 3 * NRB]
        if with_counts:
            ones_v, zb_v, cnt_s = r[7 + 3 * NRB:]
        cid = lax.axis_index("c")
        sid = lax.axis_index("s")
        wid = cid * NS + sid

        z16 = jnp.zeros((16,), jnp.float32)

        def zrow(i, _):
            for j in range(D // 16):
                rows[0][i, pl.ds(j * 16, 16)] = z16
            return 0
        lax.fori_loop(0, C, zrow, 0)

        nfull = RPT // C
        rem = RPT - nfull * C
        for k in range(nfull):
            pltpu.sync_copy(rows[0], acc_s.at[pl.ds(sid * RPT + k * C, C)])
        if rem:
            pltpu.sync_copy(rows[0].at[pl.ds(0, rem)],
                            acc_s.at[pl.ds(sid * RPT + nfull * C, rem)])

        if with_counts:
            o16 = jnp.ones((16,), jnp.float32)

            def frow(i, _):
                ones_v[i, :] = o16
                zb_v[i, :] = z16
                return 0
            lax.fori_loop(0, C, frow, 0)
            for k in range(nfull):
                pltpu.sync_copy(zb_v, cnt_s.at[pl.ds(sid * RPT + k * C, C)])
            if rem:
                pltpu.sync_copy(zb_v.at[pl.ds(0, rem)],
                                cnt_s.at[pl.ds(sid * RPT + nfull * C, rem)])

        plsc.subcore_barrier()

        def fetch_idx(batch, par):
            pltpu.async_copy(src_hbm.at[wid, pl.ds(batch * IB, IB)],
                             src_i[par], isem[par])
            pltpu.async_copy(dst_hbm.at[wid, pl.ds(batch * IB, IB)],
                             dst_i[par], isem[par])

        def wait_idx(par):
            pltpu.make_async_copy(src_hbm.at[wid, pl.ds(0, IB)],
                                  src_i[par], isem[par]).wait()
            pltpu.make_async_copy(dst_hbm.at[wid, pl.ds(0, IB)],
                                  dst_i[par], isem[par]).wait()

        def fire_g(jp):
            pltpu.async_copy(x_hbm.at[src_i[(jp // IB) % 2].at[jp % IB]],
                             rows[jp % NRB], gsem[jp % NRB])

        def wait_g(jp):
            pltpu.make_async_copy(
                x_hbm.at[src_i[(jp // IB) % 2].at[jp % IB]],
                rows[jp % NRB], gsem[jp % NRB]).wait()

        def fire_s(jp):
            idx = dst_i[(jp // IB) % 2].at[jp % IB]
            pltpu.async_copy(rows[jp % NRB], acc_s.at[idx],
                             ssem[jp % NRB], add=True)
            if with_counts:
                pltpu.async_copy(ones_v, cnt_s.at[idx],
                                 ssem[jp % NRB], add=True)

        def wait_s(jp):
            pltpu.make_async_copy(rows[jp % NRB], acc_s.at[dst_i[0].at[0]],
                                  ssem[jp % NRB]).wait()
            if with_counts:
                pltpu.make_async_copy(ones_v, cnt_s.at[dst_i[0].at[0]],
                                      ssem[jp % NRB]).wait()

        fetch_idx(0, 0)
        wait_idx(0)
        fire_g(0)
        fire_g(1)
        fire_g(2)

        def pair(mm, first, last):
            for jj in range(8):
                wait_g(jj)
                fire_s(jj)
                if not (first and jj == 0):
                    wait_s((jj - 1) % 8)
                if jj == 0:
                    fetch_idx(2 * mm + 1, 1)
                if jj == 1:
                    wait_idx(1)
                if jj == 4 and not last:
                    fetch_idx(2 * mm + 2, 0)
                if jj == 5 and not last:
                    wait_idx(0)
                if jj + 3 < 8:
                    fire_g(jj + 3)
                elif not last:
                    fire_g(jj - 5)

        pair(0, True, False)

        def pair_loop(mm, _):
            pair(mm, False, False)
            return 0
        lax.fori_loop(1, NPAIR - 1, pair_loop, 0)
        pair(NPAIR - 1, False, True)
        wait_s(7)

        plsc.subcore_barrier()

        pltpu.sync_copy(acc_s.at[pl.ds(sid * RPT, RPT)],
                        out_sums.at[cid, pl.ds(sid * RPT, RPT)])
        if with_counts:
            pltpu.sync_copy(cnt_s.at[pl.ds(sid * RPT, RPT)],
                            out_cnts.at[cid, pl.ds(sid * RPT, RPT)])

    return agg


_agg_counts = _make_agg(True)
_agg_plain = _make_agg(False)

BN = 1000


def _make_dense(relu: bool):
    def body(p_ref, c_ref, x_ref, wl_ref, b_ref, wr_ref, o_ref):
        s = p_ref[0] + p_ref[1]
        cnt = jnp.maximum(c_ref[0, :, 0:1] + c_ref[1, :, 0:1], 1.0)
        agg = s / cnt
        y = jnp.dot(agg, wl_ref[...], preferred_element_type=jnp.float32)
        y = y + jnp.dot(x_ref[...], wr_ref[...],
                        preferred_element_type=jnp.float32)
        y = y + b_ref[...]
        if relu:
            y = jnp.maximum(y, 0.0)
        o_ref[...] = y

    return pl.pallas_call(
        body,
        grid=(N // BN,),
        in_specs=[
            pl.BlockSpec((NC, BN, D), lambda i: (0, i, 0)),
            pl.BlockSpec((NC, BN, 16), lambda i: (0, i, 0)),
            pl.BlockSpec((BN, D), lambda i: (i, 0)),
            pl.BlockSpec((D, D), lambda i: (0, 0)),
            pl.BlockSpec((1, D), lambda i: (0, 0)),
            pl.BlockSpec((D, D), lambda i: (0, 0)),
        ],
        out_specs=pl.BlockSpec((BN, D), lambda i: (i, 0)),
        out_shape=jax.ShapeDtypeStruct((N, D), jnp.float32),
    )


_dense_relu = _make_dense(True)
_dense_out = _make_dense(False)


def _pad_edges(edge_index):
    src, dst = edge_index[0], edge_index[1]
    pad = EP - E
    ar = jnp.arange(pad, dtype=jnp.int32)
    pad_src = (ar * 37) % N
    pad_dst = N + ar % (NP - N)
    src_p = jnp.concatenate([src, pad_src]).reshape(NW, NCHUNK, C)
    dst_p = jnp.concatenate([dst, pad_dst]).reshape(NW, NCHUNK, C)
    return src_p, dst_p


def kernel(x_user, x_item, edge_index_u2i, edge_index_i2u,
           W1l_u2i, b1l_u2i, W1r_u2i, W1l_i2u, b1l_i2u, W1r_i2u,
           W2l_u2i, b2l_u2i, W2r_u2i, W2l_i2u, b2l_i2u, W2r_i2u):
    src_u2i, dst_u2i = _pad_edges(edge_index_u2i)
    src_i2u, dst_i2u = _pad_edges(edge_index_i2u)
    b1l_u2i = b1l_u2i.reshape(1, D)
    b1l_i2u = b1l_i2u.reshape(1, D)
    b2l_u2i = b2l_u2i.reshape(1, D)
    b2l_i2u = b2l_i2u.reshape(1, D)

    sums1_i, cnts_i = _agg_counts(x_user, src_u2i, dst_u2i)
    sums1_u, cnts_u = _agg_counts(x_item, src_i2u, dst_i2u)
    h_item = _dense_relu(sums1_i, cnts_i, x_item, W1l_u2i, b1l_u2i, W1r_u2i)
    h_user = _dense_relu(sums1_u, cnts_u, x_user, W1l_i2u, b1l_i2u, W1r_i2u)
    (sums2_i,) = _agg_plain(h_user, src_u2i, dst_u2i)
    (sums2_u,) = _agg_plain(h_item, src_i2u, dst_i2u)
    o_item = _dense_out(sums2_i, cnts_i, h_item, W2l_u2i, b2l_u2i, W2r_u2i)
    o_user = _dense_out(sums2_u, cnts_u, h_user, W2l_i2u, b2l_i2u, W2r_i2u)
    return (o_user, o_item)
